# baseline (speedup 1.0000x reference)
import functools

import jax
import jax.numpy as jnp
from jax import lax
from jax.experimental import pallas as pl
from jax.experimental.pallas import tpu as pltpu
from jax.experimental.pallas import tpu_sc as plsc

_N = 10000
_E = 320000
_D = 128
_H = 128
_C = 72
_EPS = 1e-5

_NC = 2
_NS = 16
_NW = _NC * _NS
_EPT = _E // _NW
_K = 120
_NCHUNK = 84
_PAD = _NCHUNK * _K - _EPT
_NACC = _N + 8
_RPT = _N // _NS

_BR = 2000
_G = _N // _BR


def _deg_body(col_hbm, out_hbm, col_v, hist_v):
    c = lax.axis_index("c")
    s = lax.axis_index("s")
    wid = c * _NS + s
    pltpu.sync_copy(col_hbm.at[pl.ds(wid * _EPT, _EPT)], col_v)

    def zero_body(i, carry):
        hist_v[pl.ds(i * 16, 16)] = jnp.zeros((16,), jnp.float32)
        return carry

    lax.fori_loop(0, _N // 16, zero_body, 0)

    ones = jnp.ones((16,), jnp.float32)

    def body(i, carry):
        idx = col_v[pl.ds(i * 16, 16)]
        plsc.addupdate_scatter(hist_v, [idx], ones)
        return carry

    lax.fori_loop(0, _EPT // 16, body, 0)
    pltpu.sync_copy(hist_v, out_hbm.at[pl.ds(wid * _N, _N)])


@functools.cache
def _deg_call():
    return pl.kernel(
        _deg_body,
        out_type=jax.ShapeDtypeStruct((_NW * _N,), jnp.float32),
        mesh=plsc.VectorSubcoreMesh(core_axis_name="c", subcore_axis_name="s",
                                    num_cores=_NC, num_subcores=_NS),
        scratch_types=[
            pltpu.VMEM((_EPT,), jnp.int32),
            pltpu.VMEM((_N,), jnp.float32),
        ],
        compiler_params=pltpu.CompilerParams(needs_layout_passes=False),
    )


def _spmm_body(rc_hbm, hs_hbm, z_hbm, out_hbm,
               r0, r1, r2, x0, x1, x2, x3, acc_sh,
               g0, g1, g2, xs0, xs1, xs2, xs3, s0, s1, s2):
    c = lax.axis_index("c")
    s = lax.axis_index("s")
    wid = c * _NS + s
    cbase = wid * _NCHUNK
    R, G, S = [r0, r1, r2], [g0, g1, g2], [s0, s1, s2]
    X, XS = [x0, x1, x2, x3], [xs0, xs1, xs2, xs3]

    def fire_idx(i, xsl):
        pltpu.async_copy(rc_hbm.at[cbase + i], X[xsl], XS[xsl])

    def wait_idx(i, xsl):
        pltpu.make_async_copy(rc_hbm.at[cbase + i], X[xsl], XS[xsl]).wait()

    def fire_gather(sl, xsl):
        pltpu.async_copy(hs_hbm.at[X[xsl].at[0]], R[sl], G[sl])

    def wait_gather(sl, xsl):
        pltpu.make_async_copy(hs_hbm.at[X[xsl].at[0]], R[sl], G[sl]).wait()

    def fire_scatter(sl, xsl):
        pltpu.async_copy(R[sl], acc_sh.at[X[xsl].at[1]], S[sl], add=True)

    def wait_scatter(sl, xsl):
        pltpu.make_async_copy(R[sl], acc_sh.at[X[xsl].at[1]], S[sl]).wait()

    def step(i, u, wait_s, do_idx, do_g):
        if wait_s:
            wait_scatter((u + 1) % 3, (u + 2) % 4)
        if do_idx:
            fire_idx(i + 2, (u + 2) % 4)
        if do_g:
            wait_idx(i + 1, (u + 1) % 4)
            fire_gather((u + 1) % 3, (u + 1) % 4)
        wait_gather(u % 3, u % 4)
        fire_scatter(u % 3, u % 4)

    fire_idx(0, 0)
    fire_idx(1, 1)
    pltpu.sync_copy(z_hbm, acc_sh.at[pl.ds(s * _RPT, _RPT)])
    wait_idx(0, 0)
    fire_gather(0, 0)
    plsc.subcore_barrier()

    for u in range(12):
        step(u, u, u >= 2, True, True)

    def body(t, carry):
        i0 = 12 * t
        for u in range(12):
            step(i0 + u, u, True, True, True)
        return carry

    lax.fori_loop(1, _NCHUNK // 12 - 1, body, 0)

    for u in range(12):
        step(_NCHUNK - 12 + u, u, True, u <= 9, u <= 10)
    wait_scatter(1, 2)
    wait_scatter(2, 3)
    plsc.subcore_barrier()
    pltpu.sync_copy(acc_sh.at[pl.ds(s * _RPT, _RPT)], out_hbm.at[wid])


@functools.cache
def _spmm_call():
    return pl.kernel(
        _spmm_body,
        out_type=jax.ShapeDtypeStruct((_NW, _RPT, _H), jnp.float32),
        mesh=plsc.VectorSubcoreMesh(core_axis_name="c", subcore_axis_name="s",
                                    num_cores=_NC, num_subcores=_NS),
        scratch_types=(
            [pltpu.VMEM((_K, _H), jnp.float32)] * 3
            + [pltpu.VMEM((2, _K), jnp.int32)] * 4
            + [pltpu.VMEM_SHARED((_NACC, _H), jnp.float32)]
            + [pltpu.SemaphoreType.DMA] * 10
        ),
        compiler_params=pltpu.CompilerParams(needs_layout_passes=False),
    )


def _enc_body(x_ref, degt_ref, we_ref, be_ref, wg0_ref, hs0_ref, dinvb_ref):
    h = jnp.dot(x_ref[...], we_ref[...], preferred_element_type=jnp.float32)
    h = jnp.maximum(h + be_ref[...], 0.0)
    deg = jnp.sum(degt_ref[...], axis=1, keepdims=True) + 1.0
    dinvb = jnp.broadcast_to(lax.rsqrt(deg), (_BR, _H))
    dinvb_ref[...] = dinvb
    hw = jnp.dot(h, wg0_ref[...], preferred_element_type=jnp.float32)
    hs0_ref[...] = hw * dinvb


_row_spec = pl.BlockSpec((_BR, _H), lambda i: (i, 0))
_w_spec = pl.BlockSpec((_H, _H), lambda i: (0, 0))
_b_spec = pl.BlockSpec((1, _H), lambda i: (0, 0))

_enc_call = pl.pallas_call(
    _enc_body,
    grid=(_G,),
    in_specs=[
        pl.BlockSpec((_BR, _D), lambda i: (i, 0)),
        pl.BlockSpec((_BR, _NW), lambda i: (i, 0)),
        _w_spec, _b_spec, _w_spec,
    ],
    out_specs=[_row_spec, _row_spec],
    out_shape=[
        jax.ShapeDtypeStruct((_N, _H), jnp.float32),
        jax.ShapeDtypeStruct((_N, _H), jnp.float32),
    ],
)


def _layer_body(residual, pa_ref, pb_ref, hs_ref, dinvb_ref, b_ref,
                scale_ref, beta_ref, hprev_ref, wnext_ref,
                h_ref, hsnext_ref):
    agg = pa_ref[...] + pb_ref[...] + hs_ref[...]
    conv = agg * dinvb_ref[...] + b_ref[...]
    hn = jnp.maximum(conv * scale_ref[...] + beta_ref[...], 0.0)
    h = hn + hprev_ref[...] if residual else hn
    h_ref[...] = h
    hw = jnp.dot(h, wnext_ref[...], preferred_element_type=jnp.float32)
    hsnext_ref[...] = hw * dinvb_ref[...]


def _make_layer_call(residual):
    return pl.pallas_call(
        functools.partial(_layer_body, residual),
        grid=(_G,),
        in_specs=[
            pl.BlockSpec((_BR, _H), lambda i: (i, 0)),
            pl.BlockSpec((_BR, _H), lambda i: (i + _G, 0)),
            _row_spec, _row_spec, _b_spec, _b_spec, _b_spec,
            _row_spec, _w_spec,
        ],
        out_specs=[_row_spec, _row_spec],
        out_shape=[
            jax.ShapeDtypeStruct((_N, _H), jnp.float32),
            jax.ShapeDtypeStruct((_N, _H), jnp.float32),
        ],
    )


_layer0_call = _make_layer_call(False)
_layer1_call = _make_layer_call(True)


def _final_body(pa_ref, pb_ref, hs_ref, dinvb_ref, b_ref, scale_ref,
                beta_ref, hprev_ref, wc1_ref, bc1_ref, wc2_ref, bc2_ref,
                out_ref):
    agg = pa_ref[...] + pb_ref[...] + hs_ref[...]
    conv = agg * dinvb_ref[...] + b_ref[...]
    hn = jnp.maximum(conv * scale_ref[...] + beta_ref[...], 0.0)
    h = hn + hprev_ref[...]
    t = jnp.dot(h, wc1_ref[...], preferred_element_type=jnp.float32)
    t = jnp.maximum(t + bc1_ref[...], 0.0)
    out_ref[...] = jnp.dot(t, wc2_ref[...],
                           preferred_element_type=jnp.float32) + bc2_ref[...]


_final_call = pl.pallas_call(
    _final_body,
    grid=(_G,),
    in_specs=[
        pl.BlockSpec((_BR, _H), lambda i: (i, 0)),
        pl.BlockSpec((_BR, _H), lambda i: (i + _G, 0)),
        _row_spec, _row_spec, _b_spec, _b_spec, _b_spec,
        _row_spec, _w_spec, _b_spec, _w_spec, _b_spec,
    ],
    out_specs=_row_spec,
    out_shape=jax.ShapeDtypeStruct((_N, _H), jnp.float32),
)


def kernel(x, edge_index, W_enc, b_enc, Wg0, bg0, g0, be0,
           Wg1, bg1, g1, be1, Wg2, bg2, g2, be2, Wc1, bc1, Wc2, bc2):
    row = edge_index[0]
    col = edge_index[1]
    row_p = jnp.concatenate(
        [row.reshape(_NW, _EPT), jnp.zeros((_NW, _PAD), jnp.int32)], axis=1)
    col_p = jnp.concatenate(
        [col.reshape(_NW, _EPT), jnp.full((_NW, _PAD), _N, jnp.int32)], axis=1)
    rc = jnp.stack([row_p.reshape(_NW, _NCHUNK, _K),
                    col_p.reshape(_NW, _NCHUNK, _K)], axis=2)
    rc = rc.reshape(_NW * _NCHUNK, 2, _K)
    z = jnp.zeros((_RPT, _H), jnp.float32)

    bn_scale = 1.0 / jnp.sqrt(jnp.float32(1.0 + _EPS))
    b_enc2 = b_enc.reshape(1, _H)
    bg = [b.reshape(1, _H) for b in (bg0, bg1, bg2)]
    sc = [(g * bn_scale).reshape(1, _H) for g in (g0, g1, g2)]
    be = [b.reshape(1, _H) for b in (be0, be1, be2)]

    wc1p = jnp.zeros((_H, _H), jnp.float32).at[:, : _H // 2].set(Wc1)
    bc1p = jnp.zeros((1, _H), jnp.float32).at[0, : _H // 2].set(bc1)
    wc2p = jnp.zeros((_H, _H), jnp.float32).at[: _H // 2, :_C].set(Wc2)
    bc2p = jnp.zeros((1, _H), jnp.float32).at[0, :_C].set(bc2)

    deg_parts = _deg_call()(col)
    degt = deg_parts.reshape(_NW, _N).T

    spmm = _spmm_call()
    hs0, dinvb = _enc_call(x, degt, W_enc, b_enc2, Wg0)
    p0 = spmm(rc, hs0, z).reshape(_NC * _N, _H)
    h1, hs1 = _layer0_call(p0, p0, hs0, dinvb, bg[0], sc[0], be[0], hs0, Wg1)
    p1 = spmm(rc, hs1, z).reshape(_NC * _N, _H)
    h2, hs2 = _layer1_call(p1, p1, hs1, dinvb, bg[1], sc[1], be[1], h1, Wg2)
    p2 = spmm(rc, hs2, z).reshape(_NC * _N, _H)
    out = _final_call(p2, p2, hs2, dinvb, bg[2], sc[2], be[2], h2,
                      wc1p, bc1p, wc2p, bc2p)
    return out[:, :_C]


# speedup vs baseline: 1.5781x; 1.5781x over previous
import functools

import jax
import jax.numpy as jnp
from jax import lax
from jax.experimental import pallas as pl
from jax.experimental.pallas import tpu as pltpu
from jax.experimental.pallas import tpu_sc as plsc

_N = 10000
_E = 320000
_D = 128
_H = 128
_C = 72
_EPS = 1e-5

_NC = 2
_NS = 16
_NW = _NC * _NS
_EPT = _E // _NW
_K = 125
_NCHUNK = _EPT // _K
_RPT = _N // _NS

_BR = 2000
_G = _N // _BR


def _deg_body(col_hbm, out_hbm, col_v, hist_v):
    c = lax.axis_index("c")
    s = lax.axis_index("s")
    wid = c * _NS + s
    pltpu.sync_copy(col_hbm.at[pl.ds(wid * _EPT, _EPT)], col_v)

    def zero_body(i, carry):
        hist_v[pl.ds(i * 16, 16)] = jnp.zeros((16,), jnp.float32)
        return carry

    lax.fori_loop(0, _N // 16, zero_body, 0)

    ones = jnp.ones((16,), jnp.float32)

    def body(i, carry):
        idx = col_v[pl.ds(i * 16, 16)]
        plsc.addupdate_scatter(hist_v, [idx], ones)
        return carry

    lax.fori_loop(0, _EPT // 16, body, 0)
    pltpu.sync_copy(hist_v, out_hbm.at[pl.ds(wid * _N, _N)])


@functools.cache
def _deg_call():
    return pl.kernel(
        _deg_body,
        out_type=jax.ShapeDtypeStruct((_NW * _N,), jnp.float32),
        mesh=plsc.VectorSubcoreMesh(core_axis_name="c", subcore_axis_name="s",
                                    num_cores=_NC, num_subcores=_NS),
        scratch_types=[
            pltpu.VMEM((_EPT,), jnp.int32),
            pltpu.VMEM((_N,), jnp.float32),
        ],
        compiler_params=pltpu.CompilerParams(needs_layout_passes=False),
    )


def _spmm_body(row2_hbm, col3_hbm, hs_hbm, z_hbm, out_hbm,
               row2_v, rows_a, rows_b, ci_a, ci_b, acc_sh,
               sem_a, sem_b, sem_ca, sem_cb):
    c = lax.axis_index("c")
    s = lax.axis_index("s")
    wid = c * _NS + s
    cbase = wid * _NCHUNK
    pltpu.sync_copy(row2_hbm.at[pl.ds(wid * _NCHUNK, _NCHUNK)], row2_v)
    pltpu.async_copy(hs_hbm.at[row2_v.at[0]], rows_a, sem_a)
    pltpu.async_copy(col3_hbm.at[cbase], ci_a, sem_ca)
    pltpu.sync_copy(z_hbm, acc_sh.at[pl.ds(s * _RPT, _RPT)])
    plsc.subcore_barrier()


    def body(j, carry):
        c0 = 2 * j
        c1 = c0 + 1
        pltpu.async_copy(hs_hbm.at[row2_v.at[c1]], rows_b, sem_b)
        pltpu.async_copy(col3_hbm.at[cbase + c1], ci_b, sem_cb)
        pltpu.make_async_copy(hs_hbm.at[row2_v.at[c0]], rows_a, sem_a).wait()
        pltpu.make_async_copy(col3_hbm.at[cbase], ci_a, sem_ca).wait()
        pltpu.sync_copy(rows_a, acc_sh.at[ci_a.at[0]], add=True)
        nxt = jnp.minimum(c0 + 2, _NCHUNK - 2)
        pltpu.async_copy(hs_hbm.at[row2_v.at[nxt]], rows_a, sem_a)
        pltpu.async_copy(col3_hbm.at[cbase + nxt], ci_a, sem_ca)
        pltpu.make_async_copy(hs_hbm.at[row2_v.at[c1]], rows_b, sem_b).wait()
        pltpu.make_async_copy(col3_hbm.at[cbase + c1], ci_b, sem_cb).wait()
        pltpu.sync_copy(rows_b, acc_sh.at[ci_b.at[0]], add=True)
        return carry

    lax.fori_loop(0, _NCHUNK // 2, body, 0)
    pltpu.make_async_copy(hs_hbm.at[row2_v.at[0]], rows_a, sem_a).wait()
    pltpu.make_async_copy(col3_hbm.at[cbase], ci_a, sem_ca).wait()
    plsc.subcore_barrier()
    pltpu.sync_copy(acc_sh.at[pl.ds(s * _RPT, _RPT)], out_hbm.at[wid])


@functools.cache
def _spmm_call():
    return pl.kernel(
        _spmm_body,
        out_type=jax.ShapeDtypeStruct((_NW, _RPT, _H), jnp.float32),
        mesh=plsc.VectorSubcoreMesh(core_axis_name="c", subcore_axis_name="s",
                                    num_cores=_NC, num_subcores=_NS),
        scratch_types=[
            pltpu.VMEM((_NCHUNK, _K), jnp.int32),
            pltpu.VMEM((_K, _H), jnp.float32),
            pltpu.VMEM((_K, _H), jnp.float32),
            pltpu.VMEM((1, _K), jnp.int32),
            pltpu.VMEM((1, _K), jnp.int32),
            pltpu.VMEM_SHARED((_N, _H), jnp.float32),
            pltpu.SemaphoreType.DMA,
            pltpu.SemaphoreType.DMA,
            pltpu.SemaphoreType.DMA,
            pltpu.SemaphoreType.DMA,
        ],
        compiler_params=pltpu.CompilerParams(needs_layout_passes=False),
    )


def _enc_body(x_ref, degt_ref, we_ref, be_ref, wg0_ref, hs0_ref, dinvb_ref):
    h = jnp.dot(x_ref[...], we_ref[...], preferred_element_type=jnp.float32)
    h = jnp.maximum(h + be_ref[...], 0.0)
    deg = jnp.sum(degt_ref[...], axis=1, keepdims=True) + 1.0
    dinvb = jnp.broadcast_to(lax.rsqrt(deg), (_BR, _H))
    dinvb_ref[...] = dinvb
    hw = jnp.dot(h, wg0_ref[...], preferred_element_type=jnp.float32)
    hs0_ref[...] = hw * dinvb


_row_spec = pl.BlockSpec((_BR, _H), lambda i: (i, 0))
_w_spec = pl.BlockSpec((_H, _H), lambda i: (0, 0))
_b_spec = pl.BlockSpec((1, _H), lambda i: (0, 0))

_enc_call = pl.pallas_call(
    _enc_body,
    grid=(_G,),
    in_specs=[
        pl.BlockSpec((_BR, _D), lambda i: (i, 0)),
        pl.BlockSpec((_BR, _NW), lambda i: (i, 0)),
        _w_spec, _b_spec, _w_spec,
    ],
    out_specs=[_row_spec, _row_spec],
    out_shape=[
        jax.ShapeDtypeStruct((_N, _H), jnp.float32),
        jax.ShapeDtypeStruct((_N, _H), jnp.float32),
    ],
)


def _layer_body(residual, pa_ref, pb_ref, hs_ref, dinvb_ref, b_ref,
                scale_ref, beta_ref, hprev_ref, wnext_ref,
                h_ref, hsnext_ref):
    agg = pa_ref[...] + pb_ref[...] + hs_ref[...]
    conv = agg * dinvb_ref[...] + b_ref[...]
    hn = jnp.maximum(conv * scale_ref[...] + beta_ref[...], 0.0)
    h = hn + hprev_ref[...] if residual else hn
    h_ref[...] = h
    hw = jnp.dot(h, wnext_ref[...], preferred_element_type=jnp.float32)
    hsnext_ref[...] = hw * dinvb_ref[...]


def _make_layer_call(residual):
    return pl.pallas_call(
        functools.partial(_layer_body, residual),
        grid=(_G,),
        in_specs=[
            pl.BlockSpec((_BR, _H), lambda i: (i, 0)),
            pl.BlockSpec((_BR, _H), lambda i: (i + _G, 0)),
            _row_spec, _row_spec, _b_spec, _b_spec, _b_spec,
            _row_spec, _w_spec,
        ],
        out_specs=[_row_spec, _row_spec],
        out_shape=[
            jax.ShapeDtypeStruct((_N, _H), jnp.float32),
            jax.ShapeDtypeStruct((_N, _H), jnp.float32),
        ],
    )


_layer0_call = _make_layer_call(False)
_layer1_call = _make_layer_call(True)


def _final_body(pa_ref, pb_ref, hs_ref, dinvb_ref, b_ref, scale_ref,
                beta_ref, hprev_ref, wc1_ref, bc1_ref, wc2_ref, bc2_ref,
                out_ref):
    agg = pa_ref[...] + pb_ref[...] + hs_ref[...]
    conv = agg * dinvb_ref[...] + b_ref[...]
    hn = jnp.maximum(conv * scale_ref[...] + beta_ref[...], 0.0)
    h = hn + hprev_ref[...]
    t = jnp.dot(h, wc1_ref[...], preferred_element_type=jnp.float32)
    t = jnp.maximum(t + bc1_ref[...], 0.0)
    out_ref[...] = jnp.dot(t, wc2_ref[...],
                           preferred_element_type=jnp.float32) + bc2_ref[...]


_final_call = pl.pallas_call(
    _final_body,
    grid=(_G,),
    in_specs=[
        pl.BlockSpec((_BR, _H), lambda i: (i, 0)),
        pl.BlockSpec((_BR, _H), lambda i: (i + _G, 0)),
        _row_spec, _row_spec, _b_spec, _b_spec, _b_spec,
        _row_spec, _w_spec, _b_spec, _w_spec, _b_spec,
    ],
    out_specs=_row_spec,
    out_shape=jax.ShapeDtypeStruct((_N, _H), jnp.float32),
)


def kernel(x, edge_index, W_enc, b_enc, Wg0, bg0, g0, be0,
           Wg1, bg1, g1, be1, Wg2, bg2, g2, be2, Wc1, bc1, Wc2, bc2):
    row = edge_index[0]
    col = edge_index[1]
    row2 = row.reshape(_NW * _NCHUNK, _K)
    col3 = col.reshape(_NW * _NCHUNK, 1, _K)
    z = jnp.zeros((_RPT, _H), jnp.float32)

    bn_scale = 1.0 / jnp.sqrt(jnp.float32(1.0 + _EPS))
    b_enc2 = b_enc.reshape(1, _H)
    bg = [b.reshape(1, _H) for b in (bg0, bg1, bg2)]
    sc = [(g * bn_scale).reshape(1, _H) for g in (g0, g1, g2)]
    be = [b.reshape(1, _H) for b in (be0, be1, be2)]

    wc1p = jnp.zeros((_H, _H), jnp.float32).at[:, : _H // 2].set(Wc1)
    bc1p = jnp.zeros((1, _H), jnp.float32).at[0, : _H // 2].set(bc1)
    wc2p = jnp.zeros((_H, _H), jnp.float32).at[: _H // 2, :_C].set(Wc2)
    bc2p = jnp.zeros((1, _H), jnp.float32).at[0, :_C].set(bc2)

    deg_parts = _deg_call()(col)
    degt = deg_parts.reshape(_NW, _N).T

    spmm = _spmm_call()
    hs0, dinvb = _enc_call(x, degt, W_enc, b_enc2, Wg0)
    p0 = spmm(row2, col3, hs0, z).reshape(_NC * _N, _H)
    h1, hs1 = _layer0_call(p0, p0, hs0, dinvb, bg[0], sc[0], be[0], hs0, Wg1)
    p1 = spmm(row2, col3, hs1, z).reshape(_NC * _N, _H)
    h2, hs2 = _layer1_call(p1, p1, hs1, dinvb, bg[1], sc[1], be[1], h1, Wg2)
    p2 = spmm(row2, col3, hs2, z).reshape(_NC * _N, _H)
    out = _final_call(p2, p2, hs2, dinvb, bg[2], sc[2], be[2], h2,
                      wc1p, bc1p, wc2p, bc2p)
    return out[:, :_C]


# speedup vs baseline: 1.5941x; 1.0101x over previous
import functools

import jax
import jax.numpy as jnp
from jax import lax
from jax.experimental import pallas as pl
from jax.experimental.pallas import tpu as pltpu
from jax.experimental.pallas import tpu_sc as plsc

_N = 10000
_E = 320000
_D = 128
_H = 128
_C = 72
_EPS = 1e-5

_NC = 2
_NS = 16
_NW = _NC * _NS
_EPT = _E // _NW
_K = 125
_NCHUNK = _EPT // _K
_RPT = _N // _NS

_BR = 2000
_G = _N // _BR


def _deg_body(col_hbm, out_hbm, col_v, hist_v):
    c = lax.axis_index("c")
    s = lax.axis_index("s")
    wid = c * _NS + s
    pltpu.sync_copy(col_hbm.at[pl.ds(wid * _EPT, _EPT)], col_v)

    def zero_body(i, carry):
        hist_v[pl.ds(i * 16, 16)] = jnp.zeros((16,), jnp.float32)
        return carry

    lax.fori_loop(0, _N // 16, zero_body, 0)

    ones = jnp.ones((16,), jnp.float32)

    def body(i, carry):
        idx = col_v[pl.ds(i * 16, 16)]
        plsc.addupdate_scatter(hist_v, [idx], ones)
        return carry

    lax.fori_loop(0, _EPT // 16, body, 0)
    pltpu.sync_copy(hist_v, out_hbm.at[pl.ds(wid * _N, _N)])


@functools.cache
def _deg_call():
    return pl.kernel(
        _deg_body,
        out_type=jax.ShapeDtypeStruct((_NW * _N,), jnp.float32),
        mesh=plsc.VectorSubcoreMesh(core_axis_name="c", subcore_axis_name="s",
                                    num_cores=_NC, num_subcores=_NS),
        scratch_types=[
            pltpu.VMEM((_EPT,), jnp.int32),
            pltpu.VMEM((_N,), jnp.float32),
        ],
        compiler_params=pltpu.CompilerParams(needs_layout_passes=False),
    )


def _spmm_body(row2_hbm, col3_hbm, hs_hbm, z_hbm, out_hbm,
               row2_v, rows_a, rows_b, ci_a, ci_b, acc_sh,
               sem_a, sem_b, sem_ca, sem_cb):
    c = lax.axis_index("c")
    s = lax.axis_index("s")
    wid = c * _NS + s
    cbase = wid * _NCHUNK
    pltpu.sync_copy(row2_hbm.at[pl.ds(wid * _NCHUNK, _NCHUNK)], row2_v)
    pltpu.async_copy(hs_hbm.at[row2_v.at[0]], rows_a, sem_a)
    pltpu.async_copy(col3_hbm.at[cbase], ci_a, sem_ca)
    pltpu.sync_copy(z_hbm, acc_sh.at[pl.ds(s * _RPT, _RPT)])
    plsc.subcore_barrier()


    def body(j, carry):
        c0 = 2 * j
        c1 = c0 + 1
        pltpu.async_copy(hs_hbm.at[row2_v.at[c1]], rows_b, sem_b)
        pltpu.async_copy(col3_hbm.at[cbase + c1], ci_b, sem_cb)
        pltpu.make_async_copy(hs_hbm.at[row2_v.at[c0]], rows_a, sem_a).wait()
        pltpu.make_async_copy(col3_hbm.at[cbase], ci_a, sem_ca).wait()
        pltpu.sync_copy(rows_a, acc_sh.at[ci_a.at[0]], add=True)
        nxt = jnp.minimum(c0 + 2, _NCHUNK - 2)
        pltpu.async_copy(hs_hbm.at[row2_v.at[nxt]], rows_a, sem_a)
        pltpu.async_copy(col3_hbm.at[cbase + nxt], ci_a, sem_ca)
        pltpu.make_async_copy(hs_hbm.at[row2_v.at[c1]], rows_b, sem_b).wait()
        pltpu.make_async_copy(col3_hbm.at[cbase + c1], ci_b, sem_cb).wait()
        pltpu.sync_copy(rows_b, acc_sh.at[ci_b.at[0]], add=True)
        return carry

    lax.fori_loop(0, _NCHUNK // 2, body, 0)
    pltpu.make_async_copy(hs_hbm.at[row2_v.at[0]], rows_a, sem_a).wait()
    pltpu.make_async_copy(col3_hbm.at[cbase], ci_a, sem_ca).wait()
    plsc.subcore_barrier()
    pltpu.sync_copy(acc_sh.at[pl.ds(s * _RPT, _RPT)], out_hbm.at[wid])


@functools.cache
def _spmm_call():
    return pl.kernel(
        _spmm_body,
        out_type=jax.ShapeDtypeStruct((_NW, _RPT, _H), jnp.float32),
        mesh=plsc.VectorSubcoreMesh(core_axis_name="c", subcore_axis_name="s",
                                    num_cores=_NC, num_subcores=_NS),
        scratch_types=[
            pltpu.VMEM((_NCHUNK, _K), jnp.int32),
            pltpu.VMEM((_K, _H), jnp.float32),
            pltpu.VMEM((_K, _H), jnp.float32),
            pltpu.VMEM((1, _K), jnp.int32),
            pltpu.VMEM((1, _K), jnp.int32),
            pltpu.VMEM_SHARED((_N, _H), jnp.float32),
            pltpu.SemaphoreType.DMA,
            pltpu.SemaphoreType.DMA,
            pltpu.SemaphoreType.DMA,
            pltpu.SemaphoreType.DMA,
        ],
        compiler_params=pltpu.CompilerParams(needs_layout_passes=False),
    )


def _enc_body(x_ref, degt_ref, we_ref, be_ref, wg0_ref, hs0_ref, dinvb_ref):
    h = jnp.dot(x_ref[...], we_ref[...], preferred_element_type=jnp.float32)
    h = jnp.maximum(h + be_ref[...], 0.0)
    deg = jnp.sum(degt_ref[...], axis=1, keepdims=True) + 1.0
    dinvb = jnp.broadcast_to(lax.rsqrt(deg), (_BR, _H))
    dinvb_ref[...] = dinvb
    hw = jnp.dot(h, wg0_ref[...], preferred_element_type=jnp.float32)
    hs0_ref[...] = hw * dinvb


_row_spec = pl.BlockSpec((_BR, _H), lambda i: (i, 0))
_w_spec = pl.BlockSpec((_H, _H), lambda i: (0, 0))
_b_spec = pl.BlockSpec((1, _H), lambda i: (0, 0))

_enc_call = pl.pallas_call(
    _enc_body,
    grid=(_G,),
    in_specs=[
        pl.BlockSpec((_BR, _D), lambda i: (i, 0)),
        pl.BlockSpec((_BR, _NW), lambda i: (i, 0)),
        _w_spec, _b_spec, _w_spec,
    ],
    out_specs=[_row_spec, _row_spec],
    out_shape=[
        jax.ShapeDtypeStruct((_N, _H), jnp.float32),
        jax.ShapeDtypeStruct((_N, _H), jnp.float32),
    ],
)


def _layer_body(residual, pa_ref, pb_ref, hs_ref, dinvb_ref, b_ref,
                scale_ref, beta_ref, hprev_ref, wnext_ref,
                h_ref, hsnext_ref):
    agg = pa_ref[...] + pb_ref[...] + hs_ref[...]
    conv = agg * dinvb_ref[...] + b_ref[...]
    hn = jnp.maximum(conv * scale_ref[...] + beta_ref[...], 0.0)
    h = hn + hprev_ref[...] if residual else hn
    h_ref[...] = h
    hw = jnp.dot(h, wnext_ref[...], preferred_element_type=jnp.float32)
    hsnext_ref[...] = hw * dinvb_ref[...]


def _make_layer_call(residual):
    return pl.pallas_call(
        functools.partial(_layer_body, residual),
        grid=(_G,),
        in_specs=[
            pl.BlockSpec((_BR, _H), lambda i: (i, 0)),
            pl.BlockSpec((_BR, _H), lambda i: (i + _G, 0)),
            _row_spec, _row_spec, _b_spec, _b_spec, _b_spec,
            _row_spec, _w_spec,
        ],
        out_specs=[_row_spec, _row_spec],
        out_shape=[
            jax.ShapeDtypeStruct((_N, _H), jnp.float32),
            jax.ShapeDtypeStruct((_N, _H), jnp.float32),
        ],
    )


_layer0_call = _make_layer_call(False)
_layer1_call = _make_layer_call(True)


def _final_body(pa_ref, pb_ref, hs_ref, dinvb_ref, b_ref, scale_ref,
                beta_ref, hprev_ref, wc1_ref, bc1_ref, wc2_ref, bc2_ref,
                out_ref):
    agg = pa_ref[...] + pb_ref[...] + hs_ref[...]
    conv = agg * dinvb_ref[...] + b_ref[...]
    hn = jnp.maximum(conv * scale_ref[...] + beta_ref[...], 0.0)
    h = hn + hprev_ref[...]
    t = jnp.dot(h, wc1_ref[...], preferred_element_type=jnp.float32)
    t = jnp.maximum(t + bc1_ref[...], 0.0)
    out_ref[...] = jnp.dot(t, wc2_ref[...],
                           preferred_element_type=jnp.float32) + bc2_ref[...]


_final_call = pl.pallas_call(
    _final_body,
    grid=(_G,),
    in_specs=[
        pl.BlockSpec((_BR, _H), lambda i: (i, 0)),
        pl.BlockSpec((_BR, _H), lambda i: (i + _G, 0)),
        _row_spec, _row_spec, _b_spec, _b_spec, _b_spec,
        _row_spec,
        pl.BlockSpec((_H, _H // 2), lambda i: (0, 0)),
        pl.BlockSpec((1, _H // 2), lambda i: (0, 0)),
        pl.BlockSpec((_H // 2, _C), lambda i: (0, 0)),
        pl.BlockSpec((1, _C), lambda i: (0, 0)),
    ],
    out_specs=pl.BlockSpec((_BR, _C), lambda i: (i, 0)),
    out_shape=jax.ShapeDtypeStruct((_N, _C), jnp.float32),
)


def kernel(x, edge_index, W_enc, b_enc, Wg0, bg0, g0, be0,
           Wg1, bg1, g1, be1, Wg2, bg2, g2, be2, Wc1, bc1, Wc2, bc2):
    row = edge_index[0]
    col = edge_index[1]
    row2 = row.reshape(_NW * _NCHUNK, _K)
    col3 = col.reshape(_NW * _NCHUNK, 1, _K)
    z = jnp.zeros((_RPT, _H), jnp.float32)

    bn_scale = 1.0 / jnp.sqrt(jnp.float32(1.0 + _EPS))
    b_enc2 = b_enc.reshape(1, _H)
    bg = [b.reshape(1, _H) for b in (bg0, bg1, bg2)]
    sc = [(g * bn_scale).reshape(1, _H) for g in (g0, g1, g2)]
    be = [b.reshape(1, _H) for b in (be0, be1, be2)]

    deg_parts = _deg_call()(col)
    degt = deg_parts.reshape(_NW, _N).T

    spmm = _spmm_call()
    hs0, dinvb = _enc_call(x, degt, W_enc, b_enc2, Wg0)
    p0 = spmm(row2, col3, hs0, z).reshape(_NC * _N, _H)
    h1, hs1 = _layer0_call(p0, p0, hs0, dinvb, bg[0], sc[0], be[0], hs0, Wg1)
    p1 = spmm(row2, col3, hs1, z).reshape(_NC * _N, _H)
    h2, hs2 = _layer1_call(p1, p1, hs1, dinvb, bg[1], sc[1], be[1], h1, Wg2)
    p2 = spmm(row2, col3, hs2, z).reshape(_NC * _N, _H)
    return _final_call(p2, p2, hs2, dinvb, bg[2], sc[2], be[2], h2,
                       Wc1, bc1.reshape(1, -1), Wc2, bc2.reshape(1, -1))
